# baseline (speedup 1.0000x reference)
import functools

import jax
import jax.numpy as jnp
from jax import lax
from jax.experimental import pallas as pl
from jax.experimental.pallas import tpu as pltpu
from jax.experimental.pallas import tpu_sc as plsc

_BATCH = 4096
_SEQ = 200
_D = 64
_H = 128
_NW = 32
_BPW = _BATCH // _NW
_CHUNK = 100
_NCHUNK = _SEQ // _CHUNK


def _pool_body(xj_hbm, xp_hbm, table_hbm, out_hbm,
               j_v, p_v, rows_v, out_v, sem0, sem1):
    wid = lax.axis_index("s") * 2 + lax.axis_index("c")
    pltpu.sync_copy(xj_hbm.at[wid], j_v)
    pltpu.sync_copy(xp_hbm.at[wid], p_v)

    def issue(chunk, buf, sem):
        pltpu.async_copy(table_hbm.at[j_v.at[chunk]], rows_v.at[buf], sem)

    def wait(chunk, buf, sem):
        pltpu.make_async_copy(
            table_hbm.at[j_v.at[chunk]], rows_v.at[buf], sem).wait()

    issue(0, 0, sem0)
    issue(1, 1, sem1)

    neg_inf = jnp.full((16,), -jnp.inf, jnp.float32)
    nchunks = _BPW * _NCHUNK

    def reduce_chunk(row, half, buf, acc):
        def group(acc, base):
            pvec = p_v[2 * row + half, pl.ds(base, 16)]
            a0, a1, a2, a3 = acc
            for t in range(16):
                ps = pl.multiple_of(pvec[t], 16)
                j = base + t
                a0 = jnp.maximum(a0, rows_v[buf, j, pl.ds(ps, 16)])
                a1 = jnp.maximum(a1, rows_v[buf, j, pl.ds(ps + 16, 16)])
                a2 = jnp.maximum(a2, rows_v[buf, j, pl.ds(ps + 32, 16)])
                a3 = jnp.maximum(a3, rows_v[buf, j, pl.ds(ps + 48, 16)])
            return (a0, a1, a2, a3)

        acc = lax.fori_loop(0, _CHUNK // 16, lambda g, a: group(a, 16 * g),
                            acc)
        return group(acc, _CHUNK - 16)

    def body(row, carry):
        acc = (neg_inf, neg_inf, neg_inf, neg_inf)
        wait(2 * row, 0, sem0)
        acc = reduce_chunk(row, 0, 0, acc)

        @pl.when(2 * row + 2 < nchunks)
        def _():
            issue(2 * row + 2, 0, sem0)

        wait(2 * row + 1, 1, sem1)
        a0, a1, a2, a3 = reduce_chunk(row, 1, 1, acc)

        @pl.when(2 * row + 3 < nchunks)
        def _():
            issue(2 * row + 3, 1, sem1)

        out_v[row, pl.ds(0, 16)] = a0
        out_v[row, pl.ds(16, 16)] = a1
        out_v[row, pl.ds(32, 16)] = a2
        out_v[row, pl.ds(48, 16)] = a3
        return carry

    lax.fori_loop(0, _BPW, body, 0)

    pltpu.sync_copy(out_v, out_hbm.at[pl.ds(wid * _BPW, _BPW)])


_pool = functools.partial(
    pl.kernel,
    out_type=jax.ShapeDtypeStruct((_BATCH, _D), jnp.float32),
    mesh=plsc.VectorSubcoreMesh(core_axis_name="c", subcore_axis_name="s"),
    scratch_types=[
        pltpu.VMEM((_BPW * _NCHUNK, _CHUNK), jnp.int32),
        pltpu.VMEM((_BPW * _NCHUNK, _CHUNK), jnp.int32),
        pltpu.VMEM((2, _CHUNK, 2 * _D), jnp.float32),
        pltpu.VMEM((_BPW, _D), jnp.float32),
        pltpu.SemaphoreType.DMA,
        pltpu.SemaphoreType.DMA,
    ],
)(_pool_body)


def _linear_norm_body(p_ref, wt_ref, b_ref, o_ref):
    h = jnp.dot(p_ref[...], wt_ref[...],
                preferred_element_type=jnp.float32) + b_ref[...]
    nrm = jnp.sqrt(jnp.sum(h * h, axis=1, keepdims=True))
    o_ref[...] = h / jnp.maximum(nrm, 1e-12)


def kernel(x, embed_table, W, b):
    x32 = x.astype(jnp.int32)
    xj = (x32 >> 1).reshape(_NW, _BPW * _NCHUNK, _CHUNK)
    xp = ((x32 & 1) << 6).reshape(_NW, _BPW * _NCHUNK, _CHUNK)
    table2 = embed_table.reshape(embed_table.shape[0] // 2, 2 * _D)
    pooled = _pool(xj, xp, table2)

    grid = 8
    blk = _BATCH // grid
    out = pl.pallas_call(
        _linear_norm_body,
        out_shape=jax.ShapeDtypeStruct((_BATCH, _H), jnp.float32),
        grid=(grid,),
        in_specs=[
            pl.BlockSpec((blk, _D), lambda i: (i, 0)),
            pl.BlockSpec((_D, _H), lambda i: (0, 0)),
            pl.BlockSpec((1, _H), lambda i: (0, 0)),
        ],
        out_specs=pl.BlockSpec((blk, _H), lambda i: (i, 0)),
    )(pooled, W.T, b[None, :])
    return out
